# Initial kernel scaffold; baseline (speedup 1.0000x reference)
import jax
import jax.numpy as jnp
from jax.experimental import pallas as pl


def kernel(graphs, edge_index, params):
    raise NotImplementedError("write your pallas kernel here")


# speedup vs baseline: 2.2135x; 2.2135x over previous
import jax
import jax.numpy as jnp
from jax.experimental import pallas as pl

_DIMS = [(128, 256), (256, 128), (128, 64), (64, 32), (32, 2)]
_PAD_DIN = [128, 256, 128, 128, 128]
_PAD_DOUT = [256, 128, 128, 128, 128]
_BN = 1000


def _pad2(a, rows, cols):
    return jnp.pad(a, ((0, rows - a.shape[0]), (0, cols - a.shape[1])))


def _gru_stack_body(x_ref, *args):
    *w_refs, out_ref = args
    T = x_ref.shape[0]
    bn = x_ref.shape[1]
    nl = len(_DIMS)
    hs = [jnp.zeros((bn, dp), jnp.float32) for dp in _PAD_DOUT]
    for t in range(T):
        x = x_ref[t]
        for i in range(nl):
            wx, wh, whh, b = (r[...] for r in w_refs[4 * i:4 * i + 4])
            dp = _PAD_DOUT[i]
            h = hs[i]
            xw = jnp.dot(x, wx, preferred_element_type=jnp.float32) + b
            hw = jnp.dot(h, wh, preferred_element_type=jnp.float32)
            z = jax.nn.sigmoid(xw[:, :dp] + hw[:, :dp])
            r = jax.nn.sigmoid(xw[:, dp:2 * dp] + hw[:, dp:])
            h_tilde = jnp.tanh(
                xw[:, 2 * dp:]
                + jnp.dot(h * r, whh, preferred_element_type=jnp.float32))
            h_new = z * h + (1.0 - z) * h_tilde
            if i > 0:
                h_new = jnp.maximum(h_new, 0.0)
            hs[i] = h_new
            x = h_new
        out_ref[t] = hs[-1][:, :2]


def kernel(graphs, edge_index, params):
    del edge_index
    T, N, F = graphs.shape
    inputs = [graphs]
    in_specs = [pl.BlockSpec((T, _BN, F), lambda i: (0, i, 0))]
    for i, p in enumerate(params):
        _, dout = _DIMS[i]
        dip, dp = _PAD_DIN[i], _PAD_DOUT[i]
        wx = jnp.concatenate(
            [_pad2(p['Wxz'], dip, dp),
             _pad2(p['Wxr'], dip, dp),
             _pad2(p['Wxh'], dip, dp)], axis=1)
        wh = jnp.concatenate(
            [_pad2(p['Whz'], dp, dp),
             _pad2(p['Whr'], dp, dp)], axis=1)
        whh = _pad2(p['Whh'], dp, dp)
        b = jnp.concatenate([
            jnp.pad(p['bxz'] + p['bhz'], (0, dp - dout)),
            jnp.pad(p['bxr'] + p['bhr'], (0, dp - dout)),
            jnp.pad(p['bxh'] + p['bhh'], (0, dp - dout)),
        ])[None, :]
        inputs += [wx, wh, whh, b]
        in_specs += [
            pl.BlockSpec(wx.shape, lambda i: (0, 0)),
            pl.BlockSpec(wh.shape, lambda i: (0, 0)),
            pl.BlockSpec(whh.shape, lambda i: (0, 0)),
            pl.BlockSpec(b.shape, lambda i: (0, 0)),
        ]
    return pl.pallas_call(
        _gru_stack_body,
        grid=(N // _BN,),
        in_specs=in_specs,
        out_specs=pl.BlockSpec((T, _BN, 2), lambda i: (0, i, 0)),
        out_shape=jax.ShapeDtypeStruct((T, N, 2), jnp.float32),
    )(*inputs)
